# initial kernel scaffold (unmeasured)
import jax
import jax.numpy as jnp
from jax import lax
from jax.experimental import pallas as pl
from jax.experimental.pallas import tpu as pltpu

N_DEV = 16


def kernel(x, w_mat, scale_x, scale_w):
    m_per, k = x.shape
    _, n_total = w_mat.shape
    n_per = n_total // N_DEV

    def body(x_ref, w_ref, sx_ref, sw_ref, out_ref, y_ref, send_sems, recv_sems):
        my = lax.axis_index("i")

        acc = jnp.dot(x_ref[:, :], w_ref[:, :],
                      preferred_element_type=jnp.float32)
        s = sx_ref[0] * sw_ref[0]
        yv = acc * s
        yv = yv * jax.nn.sigmoid(yv)

        for j in range(N_DEV):
            y_ref[j, :, :] = yv[:, j * n_per:(j + 1) * n_per]

        for j in range(N_DEV):
            @pl.when(my == j)
            def _(j=j):
                out_ref[pl.ds(j * m_per, m_per), :] = y_ref[j, :, :]

        rdmas = []
        for j in range(N_DEV):
            rdma = pltpu.make_async_remote_copy(
                src_ref=y_ref.at[j],
                dst_ref=out_ref.at[pl.ds(my * m_per, m_per), :],
                send_sem=send_sems.at[j],
                recv_sem=recv_sems.at[my],
                device_id=(j,),
                device_id_type=pl.DeviceIdType.MESH,
            )
            rdmas.append(rdma)

            @pl.when(my != j)
            def _(rdma=rdma):
                rdma.start()

        for src in range(N_DEV):
            recv = pltpu.make_async_remote_copy(
                src_ref=y_ref.at[src],
                dst_ref=out_ref.at[pl.ds(src * m_per, m_per), :],
                send_sem=send_sems.at[src],
                recv_sem=recv_sems.at[src],
                device_id=(src,),
                device_id_type=pl.DeviceIdType.MESH,
            )

            @pl.when(my != src)
            def _(recv=recv):
                recv.wait_recv()

        for j in range(N_DEV):
            @pl.when(my != j)
            def _(rdma=rdmas[j]):
                rdma.wait_send()

    return pl.pallas_call(
        body,
        out_shape=jax.ShapeDtypeStruct((N_DEV * m_per, n_per), jnp.float32),
        in_specs=[
            pl.BlockSpec(memory_space=pltpu.VMEM),
            pl.BlockSpec(memory_space=pltpu.VMEM),
            pl.BlockSpec(memory_space=pltpu.SMEM),
            pl.BlockSpec(memory_space=pltpu.SMEM),
        ],
        out_specs=pl.BlockSpec(memory_space=pltpu.VMEM),
        scratch_shapes=[
            pltpu.VMEM((N_DEV, m_per, n_per), jnp.float32),
            pltpu.SemaphoreType.DMA((N_DEV,)),
            pltpu.SemaphoreType.DMA((N_DEV,)),
        ],
    )(x, w_mat, scale_x, scale_w)


# baseline (device time: 51966 ns/iter reference)
import jax
import jax.numpy as jnp
from jax import lax
from jax.experimental import pallas as pl
from jax.experimental.pallas import tpu as pltpu

N_DEV = 16


def kernel(x, w_mat, scale_x, scale_w):
    m_per, k = x.shape
    _, n_total = w_mat.shape
    n_per = n_total // N_DEV

    def body(x_ref, w_ref, sx_ref, sw_ref, out_ref, y_ref, send_sems, recv_sems):
        my = lax.axis_index("i")

        acc = jnp.dot(x_ref[:, :].astype(jnp.bfloat16),
                      w_ref[:, :].astype(jnp.bfloat16),
                      preferred_element_type=jnp.float32)
        s = sx_ref[0] * sw_ref[0]
        yv = acc * s
        yv = yv * jax.nn.sigmoid(yv)

        for j in range(N_DEV):
            y_ref[j, :, :] = yv[:, j * n_per:(j + 1) * n_per]

        for j in range(N_DEV):
            @pl.when(my == j)
            def _(j=j):
                out_ref[pl.ds(j * m_per, m_per), :] = y_ref[j, :, :]

        rdmas = []
        for j in range(N_DEV):
            rdma = pltpu.make_async_remote_copy(
                src_ref=y_ref.at[j],
                dst_ref=out_ref.at[pl.ds(my * m_per, m_per), :],
                send_sem=send_sems.at[j],
                recv_sem=recv_sems.at[my],
                device_id=(j,),
                device_id_type=pl.DeviceIdType.MESH,
            )
            rdmas.append(rdma)

            @pl.when(my != j)
            def _(rdma=rdma):
                rdma.start()

        for src in range(N_DEV):
            recv = pltpu.make_async_remote_copy(
                src_ref=y_ref.at[src],
                dst_ref=out_ref.at[pl.ds(src * m_per, m_per), :],
                send_sem=send_sems.at[src],
                recv_sem=recv_sems.at[src],
                device_id=(src,),
                device_id_type=pl.DeviceIdType.MESH,
            )

            @pl.when(my != src)
            def _(recv=recv):
                recv.wait_recv()

        for j in range(N_DEV):
            @pl.when(my != j)
            def _(rdma=rdmas[j]):
                rdma.wait_send()

    return pl.pallas_call(
        body,
        out_shape=jax.ShapeDtypeStruct((N_DEV * m_per, n_per), jnp.float32),
        in_specs=[
            pl.BlockSpec(memory_space=pltpu.VMEM),
            pl.BlockSpec(memory_space=pltpu.VMEM),
            pl.BlockSpec(memory_space=pltpu.SMEM),
            pl.BlockSpec(memory_space=pltpu.SMEM),
        ],
        out_specs=pl.BlockSpec(memory_space=pltpu.VMEM),
        scratch_shapes=[
            pltpu.VMEM((N_DEV, m_per, n_per), jnp.float32),
            pltpu.SemaphoreType.DMA((N_DEV,)),
            pltpu.SemaphoreType.DMA((N_DEV,)),
        ],
        compiler_params=pltpu.CompilerParams(
            vmem_limit_bytes=100 * 1024 * 1024,
        ),
    )(x, w_mat, scale_x, scale_w)


# device time: 40973 ns/iter; 1.2683x vs baseline; 1.2683x over previous
import jax
import jax.numpy as jnp
from jax import lax
from jax.experimental import pallas as pl
from jax.experimental.pallas import tpu as pltpu

N_DEV = 16


def kernel(x, w_mat, scale_x, scale_w):
    m_per, k = x.shape
    _, n_total = w_mat.shape
    n_per = n_total // N_DEV

    def body(x_ref, w_ref, sx_ref, sw_ref, out_ref,
             y_ref, rx_ref, send_sems, recv_sems):
        my = lax.axis_index("i")

        acc = jnp.dot(x_ref[:, :].astype(jnp.bfloat16),
                      w_ref[:, :].astype(jnp.bfloat16),
                      preferred_element_type=jnp.float32)
        s = sx_ref[0] * sw_ref[0]
        yv = acc * s
        yv = yv * jax.nn.sigmoid(yv)

        for j in range(N_DEV):
            y_ref[j, :, :] = yv[:, j * n_per:(j + 1) * n_per].astype(jnp.bfloat16)

        rdmas = []
        for off in range(1, N_DEV):
            tgt = lax.rem(my + off, N_DEV)
            rdma = pltpu.make_async_remote_copy(
                src_ref=y_ref.at[tgt],
                dst_ref=rx_ref.at[my],
                send_sem=send_sems.at[off],
                recv_sem=recv_sems.at[my],
                device_id=(tgt,),
                device_id_type=pl.DeviceIdType.MESH,
            )
            rdmas.append(rdma)
            rdma.start()

        for j in range(N_DEV):
            @pl.when(my == j)
            def _(j=j):
                out_ref[pl.ds(j * m_per, m_per), :] = yv[:, j * n_per:(j + 1) * n_per]

        for src in range(N_DEV):
            recv = pltpu.make_async_remote_copy(
                src_ref=y_ref.at[src],
                dst_ref=rx_ref.at[src],
                send_sem=send_sems.at[src],
                recv_sem=recv_sems.at[src],
                device_id=(src,),
                device_id_type=pl.DeviceIdType.MESH,
            )

            @pl.when(my != src)
            def _(recv=recv, src=src):
                recv.wait_recv()
                out_ref[pl.ds(src * m_per, m_per), :] = (
                    rx_ref[src, :, :].astype(jnp.float32))

        for rdma in rdmas:
            rdma.wait_send()

    return pl.pallas_call(
        body,
        out_shape=jax.ShapeDtypeStruct((N_DEV * m_per, n_per), jnp.float32),
        in_specs=[
            pl.BlockSpec(memory_space=pltpu.VMEM),
            pl.BlockSpec(memory_space=pltpu.VMEM),
            pl.BlockSpec(memory_space=pltpu.SMEM),
            pl.BlockSpec(memory_space=pltpu.SMEM),
        ],
        out_specs=pl.BlockSpec(memory_space=pltpu.VMEM),
        scratch_shapes=[
            pltpu.VMEM((N_DEV, m_per, n_per), jnp.bfloat16),
            pltpu.VMEM((N_DEV, m_per, n_per), jnp.bfloat16),
            pltpu.SemaphoreType.DMA((N_DEV,)),
            pltpu.SemaphoreType.DMA((N_DEV,)),
        ],
        compiler_params=pltpu.CompilerParams(
            vmem_limit_bytes=100 * 1024 * 1024,
        ),
    )(x, w_mat, scale_x, scale_w)


# device time: 27398 ns/iter; 1.8967x vs baseline; 1.4955x over previous
import jax
import jax.numpy as jnp
from jax import lax
from jax.experimental import pallas as pl
from jax.experimental.pallas import tpu as pltpu

N_DEV = 16
TILES_PER_CHUNK = 4
N_CHUNKS = N_DEV // TILES_PER_CHUNK
BARRIER_STEP = 1


def kernel(x, w_mat, scale_x, scale_w):
    m_per, k = x.shape
    _, n_total = w_mat.shape
    n_per = n_total // N_DEV
    n_chunk = n_per * TILES_PER_CHUNK

    def body(x_ref, w_ref, sx_ref, sw_ref, out_ref,
             xraw_ref, xb_ref, wbuf_ref, y_ref, rx_ref,
             xdma_sem, wdma_sems, send_sems, recv_sems):
        my = lax.axis_index("i")
        my_chunk = my // TILES_PER_CHUNK

        barrier_sem = pltpu.get_barrier_semaphore()
        for off in range(1, N_DEV):
            pl.semaphore_signal(
                barrier_sem, inc=1,
                device_id=(lax.rem(my + off, N_DEV),),
                device_id_type=pl.DeviceIdType.MESH,
            )

        xdma = pltpu.make_async_copy(x_ref, xraw_ref, xdma_sem)
        xdma.start()

        def w_dma(step, buf):
            c = lax.rem(my_chunk + step, N_CHUNKS)
            return pltpu.make_async_copy(
                w_ref.at[:, pl.ds(c * n_chunk, n_chunk)],
                wbuf_ref.at[buf],
                wdma_sems.at[buf],
            )

        w_dma(0, 0).start()
        s = sx_ref[0] * sw_ref[0]
        xdma.wait()
        xb_ref[:, :] = xraw_ref[:, :].astype(jnp.bfloat16)

        rdmas = []
        pending = []
        for step in range(N_CHUNKS):
            buf = step % 2
            if step + 1 < N_CHUNKS:
                w_dma(step + 1, 1 - buf).start()
            w_dma(step, buf).wait()

            acc = jnp.dot(xb_ref[:, :], wbuf_ref[buf].astype(jnp.bfloat16),
                          preferred_element_type=jnp.float32)
            yv = acc * s
            yv = yv * jax.nn.sigmoid(yv)

            c = lax.rem(my_chunk + step, N_CHUNKS)
            chunk_rdmas = []
            for i in range(TILES_PER_CHUNK):
                t = c * TILES_PER_CHUNK + i
                tile = yv[:, i * n_per:(i + 1) * n_per]
                y_ref[t, :, :] = tile.astype(jnp.bfloat16)

                @pl.when(t == my)
                def _(tile=tile):
                    out_ref[pl.ds(my * m_per, m_per), :] = tile

                rdma = pltpu.make_async_remote_copy(
                    src_ref=y_ref.at[t],
                    dst_ref=rx_ref.at[my],
                    send_sem=send_sems.at[t],
                    recv_sem=recv_sems.at[my],
                    device_id=(t,),
                    device_id_type=pl.DeviceIdType.MESH,
                )
                rdmas.append((rdma, t))
                chunk_rdmas.append((rdma, t))

            if step < BARRIER_STEP:
                pending.extend(chunk_rdmas)
                continue
            if step == BARRIER_STEP:
                pl.semaphore_wait(barrier_sem, N_DEV - 1)
                chunk_rdmas = pending + chunk_rdmas

            for rdma, t in chunk_rdmas:
                @pl.when(t != my)
                def _(rdma=rdma):
                    rdma.start()

        for off in range(1, N_DEV):
            src = lax.rem(my - off + N_DEV, N_DEV)
            recv = pltpu.make_async_remote_copy(
                src_ref=y_ref.at[src],
                dst_ref=rx_ref.at[src],
                send_sem=send_sems.at[0],
                recv_sem=recv_sems.at[src],
                device_id=(src,),
                device_id_type=pl.DeviceIdType.MESH,
            )
            recv.wait_recv()
            out_ref[pl.ds(src * m_per, m_per), :] = (
                rx_ref[src, :, :].astype(jnp.float32))

        for rdma, t in rdmas:
            @pl.when(t != my)
            def _(rdma=rdma):
                rdma.wait_send()

    return pl.pallas_call(
        body,
        out_shape=jax.ShapeDtypeStruct((N_DEV * m_per, n_per), jnp.float32),
        in_specs=[
            pl.BlockSpec(memory_space=pl.ANY),
            pl.BlockSpec(memory_space=pl.ANY),
            pl.BlockSpec(memory_space=pltpu.SMEM),
            pl.BlockSpec(memory_space=pltpu.SMEM),
        ],
        out_specs=pl.BlockSpec(memory_space=pltpu.VMEM),
        scratch_shapes=[
            pltpu.VMEM((m_per, k), jnp.float32),
            pltpu.VMEM((m_per, k), jnp.bfloat16),
            pltpu.VMEM((2, k, n_chunk), jnp.float32),
            pltpu.VMEM((N_DEV, m_per, n_per), jnp.bfloat16),
            pltpu.VMEM((N_DEV, m_per, n_per), jnp.bfloat16),
            pltpu.SemaphoreType.DMA,
            pltpu.SemaphoreType.DMA((2,)),
            pltpu.SemaphoreType.DMA((N_DEV,)),
            pltpu.SemaphoreType.DMA((N_DEV,)),
        ],
        compiler_params=pltpu.CompilerParams(
            vmem_limit_bytes=100 * 1024 * 1024,
            collective_id=0,
        ),
    )(x, w_mat, scale_x, scale_w)
